# Initial kernel scaffold; baseline (speedup 1.0000x reference)
import jax
import jax.numpy as jnp
from jax.experimental import pallas as pl


def kernel(student_id, exercise_id, q_mask, stu_emb, exer_emb, know_emb, disc_w, kimp_w, Wc, bc, Wts, bts, Wte, bte, Wtk, btk, rows_r, cols_r, vals_r, rows_w, cols_w, vals_w, rows_rf, cols_rf, vals_rf, rows_wf, cols_wf, vals_wf):
    raise NotImplementedError("write your pallas kernel here")


# speedup vs baseline: 1.0002x; 1.0002x over previous
import jax
import jax.numpy as jnp
from jax.experimental import pallas as pl

S = 10000
E = 10000
K = 512
D = 64
L = 3
N = S + E + K
TEMP = 0.8
SSL_W = 0.05


def _spmm(rows, cols, vals, x, n):
    return jax.ops.segment_sum(vals[:, None] * x[cols], rows, num_segments=n)


def _common_forward(stu_emb, exer_emb, know_emb, Wc, bc, rr, cr, vr, rw, cw, vw):
    all_emb = jnp.concatenate([stu_emb, exer_emb, know_emb], axis=0)
    embs = [all_emb]
    right = all_emb
    wrong = all_emb
    for _ in range(L):
        right = _spmm(rr, cr, vr, right, N)
        wrong = _spmm(rw, cw, vw, wrong, N)
        all_emb = jnp.concatenate([right, wrong], axis=1) @ Wc.T + bc
        embs.append(all_emb)
    out = jnp.mean(jnp.stack(embs, axis=1), axis=1)
    return out[:S], out[S:S + E], out[S + E:]


def _info_nce(v1, v2, temp):
    pos = (v1 @ v2.T) / temp
    return -jnp.mean(jnp.diagonal(jax.nn.log_softmax(pos, axis=1)))


def kernel(student_id, exercise_id, q_mask, stu_emb, exer_emb, know_emb, disc_w, kimp_w, Wc, bc, Wts, bts, Wte, bte, Wtk, btk, rows_r, cols_r, vals_r, rows_w, cols_w, vals_w, rows_rf, cols_rf, vals_rf, rows_wf, cols_wf, vals_wf):
    stu_f, exer_f, know_f = _common_forward(stu_emb, exer_emb, know_emb, Wc, bc, rows_r, cols_r, vals_r, rows_w, cols_w, vals_w)
    stu_ff, exer_ff, know_ff = _common_forward(stu_emb, exer_emb, know_emb, Wc, bc, rows_rf, cols_rf, vals_rf, rows_wf, cols_wf, vals_wf)
    extra_loss = SSL_W * (_info_nce(stu_f, stu_ff, TEMP) + _info_nce(exer_f, exer_ff, TEMP))
    student_ts = stu_f[student_id] @ Wts.T + bts
    diff_ts = exer_f[exercise_id] @ Wte.T + bte
    knowledge_ts = know_f @ Wtk.T + btk
    disc_ts = disc_w[exercise_id]
    knowledge_impact = kimp_w[exercise_id]
    return (student_ts, diff_ts, disc_ts, knowledge_ts, extra_loss, knowledge_impact)


# speedup vs baseline: 3.5121x; 3.5114x over previous
import functools

import jax
import jax.numpy as jnp
from jax import lax
from jax.experimental import pallas as pl
from jax.experimental.pallas import tpu as pltpu
from jax.experimental.pallas import tpu_sc as plsc

S = 10000
E = 10000
K = 512
D = 64
L = 3
N = S + E + K
NE = 640000
TEMP = 0.8
SSL_W = 0.05

NT = 16
CHUNK = 128
GRP = 40
EPT = NE // NT
CPT = 320
EPT_PAD = CPT * CHUNK
NGRP = CPT // GRP
NP = 20608
ROWS_PT = NP // NT

NV = 10000
NVP = 10240
RB = 128
NRB = NVP // RB


def _sc_chains_body(x0, rows_all, cols_all, vals_all, zeros_hbm, out,
                    acc, rbuf, cbuf, vbuf, gath, gsem):
    cid = lax.axis_index("c")
    sid = lax.axis_index("s")
    base = sid * ROWS_PT

    pltpu.sync_copy(zeros_hbm.at[pl.ds(base, ROWS_PT)],
                    acc.at[pl.ds(base, ROWS_PT)])
    plsc.subcore_barrier()

    def _edge_group(g, chain, xsrc):
        pltpu.sync_copy(rows_all.at[chain, sid, pl.ds(g * GRP, GRP)], rbuf)
        pltpu.sync_copy(cols_all.at[chain, sid, pl.ds(g * GRP, GRP)], cbuf)
        pltpu.sync_copy(vals_all.at[chain, sid, pl.ds(g * GRP, GRP)], vbuf)

        def _chunk(j, carry):
            pltpu.async_copy(xsrc.at[cbuf.at[j]], gath, gsem).wait()

            def _scale(e16, c2):
                vv = vbuf[j, pl.ds(e16 * 16, 16)]
                for t in range(16):
                    e = e16 * 16 + t
                    v = vv[t]
                    for q in range(4):
                        gath[e, pl.ds(q * 16, 16)] = (
                            gath[e, pl.ds(q * 16, 16)] * v)
                return c2
            lax.fori_loop(0, CHUNK // 16, _scale, 0)

            pltpu.sync_copy(gath, acc.at[rbuf.at[j]], add=True)
            return carry
        lax.fori_loop(0, GRP, _chunk, 0)

    for k in range(2):
        chain = 2 * cid + k
        for layer in range(L):
            if layer == 0:
                xsrc = x0
            else:
                xsrc = out.at[chain, layer - 1]

            def _grp(g, carry, chain=chain, xsrc=xsrc):
                _edge_group(g, chain, xsrc)
                return carry
            lax.fori_loop(0, NGRP, _grp, 0)
            plsc.subcore_barrier()

            pltpu.sync_copy(acc.at[pl.ds(base, ROWS_PT)],
                            out.at[chain, layer, pl.ds(base, ROWS_PT)])
            pltpu.sync_copy(zeros_hbm.at[pl.ds(base, ROWS_PT)],
                            acc.at[pl.ds(base, ROWS_PT)])
            plsc.subcore_barrier()


def _sc_chains(x0, rows_all, cols_all, vals_all):
    zeros_hbm = jnp.zeros((NP, D), jnp.float32)
    mesh = plsc.VectorSubcoreMesh(core_axis_name="c", subcore_axis_name="s")
    return pl.kernel(
        _sc_chains_body,
        mesh=mesh,
        compiler_params=pltpu.CompilerParams(use_tc_tiling_on_sc=False),
        out_type=jax.ShapeDtypeStruct((4, L, NP, D), jnp.float32),
        scratch_types=[
            pltpu.VMEM_SHARED((NP, D), jnp.float32),
            pltpu.VMEM((GRP, CHUNK), jnp.int32),
            pltpu.VMEM((GRP, CHUNK), jnp.int32),
            pltpu.VMEM((GRP, CHUNK), jnp.float32),
            pltpu.VMEM((CHUNK, D), jnp.float32),
            pltpu.SemaphoreType.DMA,
        ],
    )(x0, rows_all, cols_all, vals_all, zeros_hbm)


def _finalize_body(x0_ref, xr_ref, xw_ref, wc_ref, bc_ref, out_ref):
    sr = xr_ref[0] + xr_ref[1] + xr_ref[2]
    sw = xw_ref[0] + xw_ref[1] + xw_ref[2]
    wc = wc_ref[...]
    w1 = wc[:, :D]
    w2 = wc[:, D:]
    dn = (((1,), (1,)), ((), ()))
    y = (x0_ref[...]
         + lax.dot_general(sr, w1, dn, preferred_element_type=jnp.float32)
         + lax.dot_general(sw, w2, dn, preferred_element_type=jnp.float32)
         + 3.0 * bc_ref[...])
    out_ref[...] = 0.25 * y


FBLK = NP // 8


def _finalize(x0, xr, xw, wc, bc):
    return pl.pallas_call(
        _finalize_body,
        grid=(NP // FBLK,),
        in_specs=[
            pl.BlockSpec((FBLK, D), lambda i: (i, 0)),
            pl.BlockSpec((L, FBLK, D), lambda i: (0, i, 0)),
            pl.BlockSpec((L, FBLK, D), lambda i: (0, i, 0)),
            pl.BlockSpec((D, 2 * D), lambda i: (0, 0)),
            pl.BlockSpec((1, D), lambda i: (0, 0)),
        ],
        out_specs=pl.BlockSpec((FBLK, D), lambda i: (i, 0)),
        out_shape=jax.ShapeDtypeStruct((NP, D), jnp.float32),
    )(x0, xr, xw, wc, bc.reshape(1, D))


def _nce_body(v1b_ref, v2f_ref, v2b_ref, out_ref):
    i = pl.program_id(0)
    v1 = v1b_ref[...]
    v2 = v2f_ref[...]
    dn = (((1,), (1,)), ((), ()))
    scores = lax.dot_general(v1, v2, dn, preferred_element_type=jnp.float32) / TEMP
    colmask = lax.broadcasted_iota(jnp.int32, (RB, NVP), 1) < NV
    scores = jnp.where(colmask, scores, -1e30)
    m = jnp.max(scores, axis=1, keepdims=True)
    ssum = jnp.sum(jnp.exp(scores - m), axis=1, keepdims=True)
    lse = m + jnp.log(ssum)
    diag = jnp.sum(v1 * v2b_ref[...], axis=1, keepdims=True) / TEMP
    rows = i * RB + lax.broadcasted_iota(jnp.int32, (RB, 1), 0)
    contrib = jnp.where(rows < NV, lse - diag, 0.0)
    out_ref[...] = jnp.full((1, 1, RB), jnp.sum(contrib), jnp.float32)


def _info_nce_sum(v1, v2):
    v1p = jnp.pad(v1, ((0, NVP - NV), (0, 0)))
    v2p = jnp.pad(v2, ((0, NVP - NV), (0, 0)))
    partials = pl.pallas_call(
        _nce_body,
        grid=(NRB,),
        in_specs=[
            pl.BlockSpec((RB, D), lambda i: (i, 0)),
            pl.BlockSpec((NVP, D), lambda i: (0, 0)),
            pl.BlockSpec((RB, D), lambda i: (i, 0)),
        ],
        out_specs=pl.BlockSpec((1, 1, RB), lambda i: (i, 0, 0)),
        out_shape=jax.ShapeDtypeStruct((NRB, 1, RB), jnp.float32),
    )(v1p, v2p, v1p)
    return jnp.sum(partials[:, 0, 0]) / NV


def _affine_body(x_ref, w_ref, b_ref, out_ref):
    dn = (((1,), (1,)), ((), ()))
    out_ref[...] = (lax.dot_general(x_ref[...], w_ref[...], dn,
                                    preferred_element_type=jnp.float32)
                    + b_ref[...])


def _affine(x, w, b):
    m = x.shape[0]
    return pl.pallas_call(
        _affine_body,
        out_shape=jax.ShapeDtypeStruct((m, w.shape[0]), jnp.float32),
    )(x, w, b.reshape(1, -1))


def _pad_edges(a):
    return jnp.pad(a.reshape(NT, EPT), ((0, 0), (0, EPT_PAD - EPT)))


def kernel(student_id, exercise_id, q_mask, stu_emb, exer_emb, know_emb,
           disc_w, kimp_w, Wc, bc, Wts, bts, Wte, bte, Wtk, btk,
           rows_r, cols_r, vals_r, rows_w, cols_w, vals_w,
           rows_rf, cols_rf, vals_rf, rows_wf, cols_wf, vals_wf):
    x0 = jnp.concatenate([stu_emb, exer_emb, know_emb], axis=0)
    x0 = jnp.pad(x0, ((0, NP - N), (0, 0)))

    rows_all = jnp.stack([_pad_edges(rows_r), _pad_edges(rows_w),
                          _pad_edges(rows_rf), _pad_edges(rows_wf)]
                         ).reshape(4, NT, CPT, CHUNK)
    cols_all = jnp.stack([_pad_edges(cols_r), _pad_edges(cols_w),
                          _pad_edges(cols_rf), _pad_edges(cols_wf)]
                         ).reshape(4, NT, CPT, CHUNK)
    vals_all = jnp.stack([_pad_edges(vals_r), _pad_edges(vals_w),
                          _pad_edges(vals_rf), _pad_edges(vals_wf)]
                         ).reshape(4, NT, CPT, CHUNK)

    xs = _sc_chains(x0, rows_all, cols_all, vals_all)

    out1 = _finalize(x0, xs[0], xs[1], Wc, bc)
    out2 = _finalize(x0, xs[2], xs[3], Wc, bc)

    stu_f, exer_f, know_f = out1[:S], out1[S:S + E], out1[S + E:]
    stu_ff, exer_ff = out2[:S], out2[S:S + E]
    know_f = know_f[:K]

    nce = _info_nce_sum(stu_f, stu_ff) + _info_nce_sum(exer_f, exer_ff)
    extra_loss = SSL_W * nce

    student_ts = _affine(stu_f[student_id], Wts, bts)
    diff_ts = _affine(exer_f[exercise_id], Wte, bte)
    knowledge_ts = _affine(know_f, Wtk, btk)
    disc_ts = disc_w[exercise_id]
    knowledge_impact = kimp_w[exercise_id]
    return (student_ts, diff_ts, disc_ts, knowledge_ts, extra_loss,
            knowledge_impact)
